# baseline (device time: 17943 ns/iter reference)
import jax
import jax.numpy as jnp
from jax import lax
from jax.experimental import pallas as pl
from jax.experimental.pallas import tpu as pltpu

N_DEV = 4
B, SQ, SKV, D_MODEL = 2, 128, 128, 512
HQ_LOCAL, DH = 4, 64
BLK = 64
NQ = B * SQ // BLK


def _body(x_ref, wq_ref, k_ref, v_ref, wo_ref, out_ref,
          send_ref, recv_ref, send_sems, recv_sems):
    my_pos = lax.axis_index("i")
    p1 = my_pos ^ 1
    p2 = my_pos ^ 2

    barrier_sem = pltpu.get_barrier_semaphore()
    for p in (p1, p2):
        pl.semaphore_signal(
            barrier_sem, inc=1,
            device_id=(p,), device_id_type=pl.DeviceIdType.MESH,
        )

    wq = wq_ref[:]
    wo = wo_ref[:]

    def partial_for_quarter(t):
        b, blk = divmod(t, 2)
        rows = pl.ds(blk * BLK, BLK)
        q = lax.dot_general(
            x_ref[b, rows], wq, (((1,), (0,)), ((), ())),
            preferred_element_type=jnp.float32,
        )
        q = (q * 0.125).astype(jnp.bfloat16)
        pacc = None
        for h in range(HQ_LOCAL):
            k = k_ref[b, h, rows]
            v = v_ref[b, h, rows]
            s = lax.dot_general(
                q[:, h * DH:(h + 1) * DH], k, (((1,), (1,)), ((), ())),
                preferred_element_type=jnp.float32,
            )
            m = jnp.max(s, axis=-1, keepdims=True)
            w = jnp.exp(s - m)
            w = w / jnp.sum(w, axis=-1, keepdims=True)
            ctx = lax.dot_general(
                w.astype(jnp.bfloat16), v, (((1,), (0,)), ((), ())),
                preferred_element_type=jnp.float32,
            )
            p_h = lax.dot_general(
                ctx.astype(jnp.bfloat16), wo[h * DH:(h + 1) * DH],
                (((1,), (0,)), ((), ())),
                preferred_element_type=jnp.float32,
            )
            pacc = p_h if pacc is None else pacc + p_h
        return pacc

    def exchange(slot, peer):
        return pltpu.make_async_remote_copy(
            src_ref=send_ref.at[slot],
            dst_ref=recv_ref.at[slot],
            send_sem=send_sems.at[slot],
            recv_sem=recv_sems.at[slot],
            device_id=(peer,),
            device_id_type=pl.DeviceIdType.MESH,
        )

    partials, r1 = [], []
    for t in range(NQ):
        pt = partial_for_quarter(t)
        partials.append(pt)
        send_ref[t] = pt.astype(jnp.bfloat16)
        if t == 0:
            pl.semaphore_wait(barrier_sem, 2)
        r1.append(exchange(t, p1))
        r1[t].start()

    accs, r2 = [], []
    for t in range(NQ):
        r1[t].wait_recv()
        acc = partials[t] + recv_ref[t].astype(jnp.float32)
        accs.append(acc)
        send_ref[NQ + t] = acc.astype(jnp.bfloat16)
        r2.append(exchange(NQ + t, p2))
        r2[t].start()

    for t in range(NQ):
        b, blk = divmod(t, 2)
        r2[t].wait_recv()
        out_ref[b, pl.ds(blk * BLK, BLK)] = (
            accs[t] + recv_ref[NQ + t].astype(jnp.float32)
        ).astype(jnp.bfloat16)

    for t in range(NQ):
        r1[t].wait_send()
        r2[t].wait_send()


def kernel(x, Wq, K_ext, V_ext, Wo):
    my_pos = lax.axis_index("i")
    bf16 = jnp.bfloat16
    k_loc = jnp.transpose(
        lax.dynamic_slice_in_dim(
            K_ext.astype(bf16), my_pos * HQ_LOCAL, HQ_LOCAL, axis=2),
        (0, 2, 1, 3),
    )
    v_loc = jnp.transpose(
        lax.dynamic_slice_in_dim(
            V_ext.astype(bf16), my_pos * HQ_LOCAL, HQ_LOCAL, axis=2),
        (0, 2, 1, 3),
    )
    return pl.pallas_call(
        _body,
        out_shape=jax.ShapeDtypeStruct((B, SQ, D_MODEL), bf16),
        in_specs=[pl.BlockSpec(memory_space=pltpu.VMEM)] * 5,
        out_specs=pl.BlockSpec(memory_space=pltpu.VMEM),
        scratch_shapes=[
            pltpu.VMEM((2 * NQ, BLK, D_MODEL), bf16),
            pltpu.VMEM((2 * NQ, BLK, D_MODEL), bf16),
            pltpu.SemaphoreType.DMA((2 * NQ,)),
            pltpu.SemaphoreType.DMA((2 * NQ,)),
        ],
        compiler_params=pltpu.CompilerParams(collective_id=0),
    )(x.astype(bf16), Wq.astype(bf16), k_loc, v_loc, Wo.astype(bf16))


# device time: 16217 ns/iter; 1.1064x vs baseline; 1.1064x over previous
import jax
import jax.numpy as jnp
from jax import lax
from jax.experimental import pallas as pl
from jax.experimental.pallas import tpu as pltpu

N_DEV = 4
B, SQ, SKV, D_MODEL = 2, 128, 128, 512
HQ_LOCAL, DH = 4, 64
BLK = 64


def _body(x_ref, wq_ref, k_ref, v_ref, wo_ref, out_ref,
          send_ref, recv_ref, send_sems, recv_sems):
    my_pos = lax.axis_index("i")
    p1 = my_pos ^ 1
    p2 = my_pos ^ 2

    barrier_sem = pltpu.get_barrier_semaphore()
    for p in (p1, p2):
        pl.semaphore_signal(
            barrier_sem, inc=1,
            device_id=(p,), device_id_type=pl.DeviceIdType.MESH,
        )

    wq = wq_ref[:]
    wo = wo_ref[:]

    def partial_for_batch(b):
        q = lax.dot_general(
            x_ref[b], wq, (((1,), (0,)), ((), ())),
            preferred_element_type=jnp.float32,
        )
        q = (q * 0.125).astype(jnp.bfloat16)
        pacc = None
        for h in range(HQ_LOCAL):
            blocks = []
            for blk in range(2):
                rows = slice(blk * BLK, (blk + 1) * BLK)
                s = lax.dot_general(
                    q[rows, h * DH:(h + 1) * DH], k_ref[b, h, rows],
                    (((1,), (1,)), ((), ())),
                    preferred_element_type=jnp.float32,
                )
                w = jnp.exp(s)
                r = 1.0 / jnp.sum(w, axis=-1, keepdims=True)
                ctx = lax.dot_general(
                    w.astype(jnp.bfloat16), v_ref[b, h, rows],
                    (((1,), (0,)), ((), ())),
                    preferred_element_type=jnp.float32,
                )
                blocks.append((ctx * r).astype(jnp.bfloat16))
            ctx_h = jnp.concatenate(blocks, axis=0)
            p_h = lax.dot_general(
                ctx_h, wo[h * DH:(h + 1) * DH], (((1,), (0,)), ((), ())),
                preferred_element_type=jnp.float32,
            )
            pacc = p_h if pacc is None else pacc + p_h
        return pacc

    def exchange(slot, peer):
        return pltpu.make_async_remote_copy(
            src_ref=send_ref.at[slot],
            dst_ref=recv_ref.at[slot],
            send_sem=send_sems.at[slot],
            recv_sem=recv_sems.at[slot],
            device_id=(peer,),
            device_id_type=pl.DeviceIdType.MESH,
        )

    partials, r1 = [], []
    for b in range(B):
        pb = partial_for_batch(b)
        partials.append(pb)
        send_ref[b] = pb.astype(jnp.bfloat16)
        if b == 0:
            pl.semaphore_wait(barrier_sem, 2)
        r1.append(exchange(b, p1))
        r1[b].start()

    accs, r2 = [], []
    for b in range(B):
        r1[b].wait_recv()
        acc = partials[b] + recv_ref[b].astype(jnp.float32)
        accs.append(acc)
        send_ref[B + b] = acc.astype(jnp.bfloat16)
        r2.append(exchange(B + b, p2))
        r2[b].start()

    for b in range(B):
        r2[b].wait_recv()
        out_ref[b] = (accs[b] + recv_ref[B + b].astype(jnp.float32)
                      ).astype(jnp.bfloat16)

    for b in range(B):
        r1[b].wait_send()
        r2[b].wait_send()


def kernel(x, Wq, K_ext, V_ext, Wo):
    my_pos = lax.axis_index("i")
    bf16 = jnp.bfloat16
    k_loc = jnp.transpose(
        lax.dynamic_slice_in_dim(
            K_ext.astype(bf16), my_pos * HQ_LOCAL, HQ_LOCAL, axis=2),
        (0, 2, 1, 3),
    )
    v_loc = jnp.transpose(
        lax.dynamic_slice_in_dim(
            V_ext.astype(bf16), my_pos * HQ_LOCAL, HQ_LOCAL, axis=2),
        (0, 2, 1, 3),
    )
    return pl.pallas_call(
        _body,
        out_shape=jax.ShapeDtypeStruct((B, SQ, D_MODEL), bf16),
        in_specs=[pl.BlockSpec(memory_space=pltpu.VMEM)] * 5,
        out_specs=pl.BlockSpec(memory_space=pltpu.VMEM),
        scratch_shapes=[
            pltpu.VMEM((2 * B, SQ, D_MODEL), bf16),
            pltpu.VMEM((2 * B, SQ, D_MODEL), bf16),
            pltpu.SemaphoreType.DMA((2 * B,)),
            pltpu.SemaphoreType.DMA((2 * B,)),
        ],
        compiler_params=pltpu.CompilerParams(collective_id=0),
    )(x.astype(bf16), Wq.astype(bf16), k_loc, v_loc, Wo.astype(bf16))
